# Initial kernel scaffold; baseline (speedup 1.0000x reference)
import jax
import jax.numpy as jnp
from jax.experimental import pallas as pl


def kernel(x, weight):
    raise NotImplementedError("write your pallas kernel here")


# speedup vs baseline: 2.9697x; 2.9697x over previous
import functools

import jax
import jax.numpy as jnp
from jax import lax
from jax.experimental import pallas as pl
from jax.experimental.pallas import tpu as pltpu
from jax.experimental.pallas import tpu_sc as plsc

ENTRIES = 100000
DIM2 = 128
BATCH = 4096
HIST = 50
B = BATCH * HIST

NC = 2
NS = 16
NW = NC * NS
BPW = B // NW
CH = 128
NCH = BPW // CH

_mesh = plsc.VectorSubcoreMesh(core_axis_name="c", subcore_axis_name="s")


@functools.partial(
    pl.kernel,
    out_type=jax.ShapeDtypeStruct((NW, NCH, CH, DIM2), jnp.float32),
    mesh=_mesh,
    scratch_types=[
        pltpu.VMEM((NCH, CH), jnp.int32),
        pltpu.VMEM((CH, DIM2), jnp.float32),
        pltpu.SemaphoreType.DMA,
    ],
)
def _gather(idx_hbm, table_hbm, out_hbm, idx_v, rows_v, sem):
    wid = lax.axis_index("s") * NC + lax.axis_index("c")
    pltpu.sync_copy(idx_hbm.at[wid], idx_v)

    def step(j, carry):
        pltpu.async_copy(table_hbm.at[idx_v.at[j]], rows_v, sem).wait()
        pltpu.sync_copy(rows_v, out_hbm.at[wid, j])
        return carry

    lax.fori_loop(0, NCH, step, 0)


def kernel(x, weight):
    idx = x.reshape(NW, NCH, CH).astype(jnp.int32)
    out = _gather(idx, weight)
    return out.reshape(BATCH, HIST, DIM2)


# speedup vs baseline: 3.3367x; 1.1236x over previous
import functools

import jax
import jax.numpy as jnp
from jax import lax
from jax.experimental import pallas as pl
from jax.experimental.pallas import tpu as pltpu
from jax.experimental.pallas import tpu_sc as plsc

ENTRIES = 100000
DIM2 = 128
BATCH = 4096
HIST = 50
B = BATCH * HIST

NC = 2
NS = 16
NW = NC * NS
BPW = B // NW
CH = 128
NCH = BPW // CH

_mesh = plsc.VectorSubcoreMesh(core_axis_name="c", subcore_axis_name="s")


NBUF = 5


@functools.partial(
    pl.kernel,
    out_type=jax.ShapeDtypeStruct((NW, NCH, CH, DIM2), jnp.float32),
    mesh=_mesh,
    scratch_types=[
        pltpu.VMEM((NCH, CH), jnp.int32),
        [pltpu.VMEM((CH, DIM2), jnp.float32)] * NBUF,
        [pltpu.SemaphoreType.DMA] * NBUF,
        [pltpu.SemaphoreType.DMA] * NBUF,
    ],
)
def _gather(idx_hbm, table_hbm, out_hbm, idx_v, rows, gsem, wsem):
    wid = lax.axis_index("s") * NC + lax.axis_index("c")
    pltpu.sync_copy(idx_hbm.at[wid], idx_v)

    for b in range(NBUF):
        pltpu.async_copy(table_hbm.at[idx_v.at[b]], rows[b], gsem[b])

    def round_(i, carry):
        g = i * NBUF
        for b in range(NBUF):
            j = g + b
            pltpu.make_async_copy(table_hbm.at[idx_v.at[j]], rows[b],
                                  gsem[b]).wait()
            pltpu.async_copy(rows[b], out_hbm.at[wid, j], wsem[b])
            pltpu.make_async_copy(rows[b], out_hbm.at[wid, j], wsem[b]).wait()
            pltpu.async_copy(table_hbm.at[idx_v.at[j + NBUF]], rows[b],
                             gsem[b])
        return carry

    lax.fori_loop(0, NCH // NBUF - 1, round_, 0)

    for b in range(NBUF):
        j = NCH - NBUF + b
        pltpu.make_async_copy(table_hbm.at[idx_v.at[j]], rows[b],
                              gsem[b]).wait()
        pltpu.sync_copy(rows[b], out_hbm.at[wid, j])


def kernel(x, weight):
    idx = x.reshape(NW, NCH, CH).astype(jnp.int32)
    out = _gather(idx, weight)
    return out.reshape(BATCH, HIST, DIM2)


# speedup vs baseline: 5.9814x; 1.7926x over previous
import functools

import jax
import jax.numpy as jnp
from jax import lax
from jax.experimental import pallas as pl
from jax.experimental.pallas import tpu as pltpu
from jax.experimental.pallas import tpu_sc as plsc

ENTRIES = 100000
DIM2 = 128
BATCH = 4096
HIST = 50

NC = 2
NS = 16
NW = NC * NS
BPC = 2
CI = BPC * HIST
BPW = BATCH // NW
NCH = BPW // BPC
NBUF = 4

_mesh = plsc.VectorSubcoreMesh(core_axis_name="c", subcore_axis_name="s")


@functools.partial(
    pl.kernel,
    out_type=jax.ShapeDtypeStruct((BATCH, HIST, DIM2), jnp.float32),
    mesh=_mesh,
    scratch_types=[
        pltpu.VMEM((NCH, CI), jnp.int32),
        [pltpu.VMEM((CI, DIM2), jnp.float32)] * NBUF,
        [pltpu.SemaphoreType.DMA] * NBUF,
        [pltpu.SemaphoreType.DMA] * NBUF,
    ],
    compiler_params=pltpu.CompilerParams(use_tc_tiling_on_sc=True),
)
def _gather(idx_hbm, table_hbm, out_hbm, idx_v, rows, gsem, wsem):
    wid = lax.axis_index("s") * NC + lax.axis_index("c")
    base = wid * BPW
    pltpu.sync_copy(idx_hbm.at[pl.ds(wid * NCH, NCH)], idx_v)

    def out_slice(j):
        return out_hbm.at[pl.ds(base + j * BPC, BPC)]

    for b in range(NBUF):
        pltpu.async_copy(table_hbm.at[idx_v.at[b]], rows[b], gsem[b])

    def round_(i, carry):
        g = i * NBUF
        for b in range(NBUF):
            j = g + b
            pltpu.make_async_copy(table_hbm.at[idx_v.at[j]], rows[b],
                                  gsem[b]).wait()
            rv = rows[b].reshape(BPC, HIST, DIM2)
            pltpu.async_copy(rv, out_slice(j), wsem[b])
            pltpu.make_async_copy(rv, out_slice(j), wsem[b]).wait()
            pltpu.async_copy(table_hbm.at[idx_v.at[j + NBUF]], rows[b],
                             gsem[b])
        return carry

    lax.fori_loop(0, NCH // NBUF - 1, round_, 0)

    for b in range(NBUF):
        j = NCH - NBUF + b
        pltpu.make_async_copy(table_hbm.at[idx_v.at[j]], rows[b],
                              gsem[b]).wait()
        pltpu.sync_copy(rows[b].reshape(BPC, HIST, DIM2), out_slice(j))


def kernel(x, weight):
    idx = x.reshape(NW * NCH, CI).astype(jnp.int32)
    return _gather(idx, weight)
